# Initial kernel scaffold; baseline (speedup 1.0000x reference)
import jax
import jax.numpy as jnp
from jax.experimental import pallas as pl


def kernel(x, x_0, edge_index, W_pre, W1, W2, Wr, ln_gamma, ln_beta, bias):
    raise NotImplementedError("write your pallas kernel here")


# speedup vs baseline: 2.7697x; 2.7697x over previous
import functools
import math

import jax
import jax.numpy as jnp
from jax import lax
from jax.experimental import pallas as pl
from jax.experimental.pallas import tpu as pltpu
from jax.experimental.pallas import tpu_sc as plsc

_ALPHA = 0.1
_BETA = math.log(0.5 / 1 + 1)
_C1 = (1.0 - _ALPHA) * (1.0 - _BETA)
_C2 = _ALPHA * (1.0 - _BETA)

_CHUNK_NODES = 8
_NUM_WORKERS = 32


def _pre_kernel(x_ref, w_ref, z_ref):
    z = lax.dot_general(x_ref[...], w_ref[...], (((0,), (1,)), ((), ())),
                        preferred_element_type=jnp.float32)
    z_ref[...] = jnp.maximum(z, 0.0)


def _pre_matmul(x2d, w_pre, b, c, n):
    return pl.pallas_call(
        _pre_kernel,
        grid=(b,),
        in_specs=[
            pl.BlockSpec((c, n), lambda i: (i, 0)),
            pl.BlockSpec((c, c), lambda i: (0, 0)),
        ],
        out_specs=pl.BlockSpec((n, c), lambda i: (i, 0)),
        out_shape=jax.ShapeDtypeStruct((b * n, c), jnp.float32),
    )(x2d, w_pre)


def _sc_gather_max(z2d, idx2d, bn, c, k):
    num_chunks = bn // _CHUNK_NODES
    rows_per_chunk = _CHUNK_NODES * k
    niter = (num_chunks + _NUM_WORKERS - 1) // _NUM_WORKERS
    mesh = plsc.VectorSubcoreMesh(core_axis_name="c", subcore_axis_name="s")

    @functools.partial(
        pl.kernel,
        mesh=mesh,
        out_type=jax.ShapeDtypeStruct((bn, c), jnp.float32),
        scratch_types=[
            pltpu.VMEM((rows_per_chunk,), jnp.int32),
            pltpu.VMEM((rows_per_chunk, c), jnp.float32),
            pltpu.VMEM((_CHUNK_NODES, c), jnp.float32),
            pltpu.SemaphoreType.DMA,
        ],
    )
    def sc_kernel(z_hbm, idx_hbm, out_hbm, idx_v, rows_v, aggr_v, sem):
        wid = lax.axis_index("s") * 2 + lax.axis_index("c")

        def step(t, carry):
            ch = t * _NUM_WORKERS + wid

            @pl.when(ch < num_chunks)
            def _():
                pltpu.sync_copy(idx_hbm.at[ch], idx_v)
                pltpu.async_copy(z_hbm.at[idx_v], rows_v, sem).wait()

                def node_body(nl, cc):
                    base = nl * k
                    for cg in range(c // 16):
                        acc = rows_v[base, pl.ds(cg * 16, 16)]
                        for kk in range(1, k):
                            acc = jnp.maximum(
                                acc, rows_v[base + kk, pl.ds(cg * 16, 16)])
                        aggr_v[nl, pl.ds(cg * 16, 16)] = acc
                    return cc

                lax.fori_loop(0, _CHUNK_NODES, node_body, 0)
                pltpu.sync_copy(aggr_v, out_hbm.at[pl.ds(ch * _CHUNK_NODES,
                                                         _CHUNK_NODES)])

            return carry

        lax.fori_loop(0, niter, step, 0)

    return sc_kernel(z2d, idx2d)


def _combine_kernel(aggr_ref, x0_ref, x_ref, w1_ref, w2_ref, wr_ref,
                    gamma_ref, beta_ref, bias_ref, out_ref):
    c = aggr_ref.shape[1]
    rows = lax.broadcasted_iota(jnp.int32, (c, c), 0)
    cols = lax.broadcasted_iota(jnp.int32, (c, c), 1)
    eye = jnp.where(rows == cols, 1.0, 0.0).astype(jnp.float32)
    w1p = w1_ref[...] * _BETA + eye * _C1
    w2p = w2_ref[...] * _BETA + eye * _C2
    cn = (((1,), (1,)), ((), ()))
    pre = lax.dot_general(w1p, aggr_ref[...], cn,
                          preferred_element_type=jnp.float32)
    pre = pre + lax.dot_general(w2p, x0_ref[...], cn,
                                preferred_element_type=jnp.float32)
    pre = pre + lax.dot_general(wr_ref[...], x_ref[...],
                                (((1,), (0,)), ((), ())),
                                preferred_element_type=jnp.float32)
    y = jnp.maximum(pre, 0.0)
    mu = jnp.mean(y, axis=0, keepdims=True)
    var = jnp.mean(y * y, axis=0, keepdims=True) - mu * mu
    yn = (y - mu) * lax.rsqrt(var + 1e-5)
    yn = yn * gamma_ref[...] + beta_ref[...]
    yb = yn + bias_ref[...]
    nrm = jnp.sqrt(jnp.sum(yb * yb, axis=0, keepdims=True))
    out_ref[...] = yb / jnp.maximum(nrm, 1e-12)


def _combine(aggr, x0_2d, x2d, w1, w2, wr, gamma, beta, bias, b, c, n):
    grid = (b,)
    node_spec = pl.BlockSpec((n, c), lambda i: (i, 0))
    cm_spec = pl.BlockSpec((c, n), lambda i: (i, 0))
    w_spec = pl.BlockSpec((c, c), lambda i: (0, 0))
    v_spec = pl.BlockSpec((c, 1), lambda i: (0, 0))
    return pl.pallas_call(
        _combine_kernel,
        grid=grid,
        in_specs=[node_spec, node_spec, cm_spec, w_spec, w_spec, w_spec,
                  v_spec, v_spec, v_spec],
        out_specs=cm_spec,
        out_shape=jax.ShapeDtypeStruct((b * c, n), jnp.float32),
    )(aggr, x0_2d, x2d, w1, w2, wr, gamma, beta, bias)


def kernel(x, x_0, edge_index, W_pre, W1, W2, Wr, ln_gamma, ln_beta, bias):
    b, c, n = x.shape[0], x.shape[1], x.shape[2]
    k = edge_index.shape[-1]
    x2d = x.reshape(b * c, n)
    x0_2d = x_0.reshape(b * n, c)
    idx = edge_index[0]
    idx_flat = (idx + (jnp.arange(b, dtype=idx.dtype) * n)[:, None, None])
    idx2d = idx_flat.reshape(-1, _CHUNK_NODES * k)

    z = _pre_matmul(x2d, W_pre, b, c, n)
    aggr = _sc_gather_max(z, idx2d, b * n, c, k)
    out2d = _combine(aggr, x0_2d, x2d, W1, W2, Wr,
                     ln_gamma.reshape(c, 1), ln_beta.reshape(c, 1),
                     bias.reshape(c, 1), b, c, n)
    return out2d.reshape(b, c, n, 1)


# speedup vs baseline: 5.9790x; 2.1588x over previous
import functools
import math

import jax
import jax.numpy as jnp
from jax import lax
from jax.experimental import pallas as pl
from jax.experimental.pallas import tpu as pltpu
from jax.experimental.pallas import tpu_sc as plsc

_ALPHA = 0.1
_BETA = math.log(0.5 / 1 + 1)
_C1 = (1.0 - _ALPHA) * (1.0 - _BETA)
_C2 = _ALPHA * (1.0 - _BETA)

_CHUNK_NODES = 8
_NUM_WORKERS = 32
_NB = 2000
_NBUF = 4
_WIN = 512


def _pre_kernel(x_ref, w_ref, z_ref):
    z = lax.dot_general(x_ref[...], w_ref[...], (((1,), (1,)), ((), ())),
                        preferred_element_type=jnp.float32)
    z_ref[...] = jnp.maximum(z, 0.0)


def _pre_matmul(x_nm, w_pre, bw, n, c):
    nblk = n // _NB
    return pl.pallas_call(
        _pre_kernel,
        grid=(nblk,),
        in_specs=[
            pl.BlockSpec((_NB, c),
                         lambda j, bw=bw, nblk=nblk: (bw * nblk + j, 0)),
            pl.BlockSpec((c, c), lambda j: (0, 0)),
        ],
        out_specs=pl.BlockSpec((_NB, c), lambda j: (j, 0)),
        out_shape=jax.ShapeDtypeStruct((n, c), jnp.float32),
    )(x_nm, w_pre)


def _sc_gather_max(zb, e3, tail_fidx, bw, c, k, n):
    rows_per_chunk = _CHUNK_NODES * k
    nchunks = n // _CHUNK_NODES
    base = nchunks // _NUM_WORKERS
    rem = nchunks % _NUM_WORKERS
    max_cw = base + (1 if rem else 0)
    aligned_n = (n // 128) * 128
    astart_cap = aligned_n - _WIN
    nouter = (max_cw + _NBUF - 1) // _NBUF
    mesh = plsc.VectorSubcoreMesh(core_axis_name="c", subcore_axis_name="s")

    scratch = ([pltpu.VMEM((k * _WIN + 64,), jnp.int32),
                pltpu.VMEM((max_cw, rows_per_chunk), jnp.int32),
                pltpu.VMEM((8, rows_per_chunk), jnp.int32)]
               + [pltpu.VMEM((rows_per_chunk, c), jnp.float32)] * _NBUF
               + [pltpu.VMEM((_CHUNK_NODES, c), jnp.float32)] * _NBUF
               + [pltpu.SemaphoreType.DMA] * (1 + 2 * _NBUF))

    @functools.partial(
        pl.kernel,
        mesh=mesh,
        out_type=jax.ShapeDtypeStruct((n, c), jnp.float32),
        scratch_types=scratch,
        compiler_params=pltpu.CompilerParams(use_tc_tiling_on_sc=True,
                                             needs_layout_passes=False),
    )
    def sc_kernel(z_hbm, e_hbm, tail_hbm, out_hbm, idx_blk, fidx, tail_v,
                  *bufs):
        rows_v = bufs[:_NBUF]
        aggr_v = bufs[_NBUF:2 * _NBUF]
        isem = bufs[2 * _NBUF]
        gsem = bufs[2 * _NBUF + 1:2 * _NBUF + 1 + _NBUF]
        osem = bufs[2 * _NBUF + 1 + _NBUF:]
        wid = lax.axis_index("s") * 2 + lax.axis_index("c")
        is_last = wid == _NUM_WORKERS - 1
        c0 = wid * base + jnp.minimum(wid, rem)
        cnt = base + jnp.where(wid < rem, 1, 0)
        staged = cnt - jnp.where(is_last, 2, 0)
        astart = pl.multiple_of(
            jnp.minimum((c0 * _CHUNK_NODES) // 128 * 128, astart_cap), 128)
        loff = c0 * _CHUNK_NODES - astart

        for kk in range(k):
            pltpu.async_copy(e_hbm.at[bw, kk, pl.ds(astart, _WIN)],
                             idx_blk.at[pl.ds(kk * _WIN, _WIN)], isem)
        for kk in range(k):
            pltpu.make_async_copy(e_hbm.at[bw, kk, pl.ds(astart, _WIN)],
                                  idx_blk.at[pl.ds(kk * _WIN, _WIN)],
                                  isem).wait()

        lanes = lax.iota(jnp.int32, 16)
        jmod = jnp.where(lanes >= 8, lanes - 8, lanes)
        rowadd = jnp.where(lanes >= 8, 1, 0)
        colbase = jmod * 16

        def pair_body(p, carry):
            rows = jnp.broadcast_to(2 * p, (16,)).astype(jnp.int32) + rowadd
            for kk in range(k):
                v = idx_blk[pl.ds(kk * _WIN + loff + p * 16, 16)]
                plsc.store_scatter(fidx, [rows, colbase + kk], v)
            return carry

        lax.fori_loop(0, (staged + 1) // 2, pair_body, 0)

        @pl.when(is_last)
        def _():
            pltpu.sync_copy(tail_hbm.at[bw], tail_v)
            for r in range(2):
                for cg in range(rows_per_chunk // 16):
                    sl = pl.ds(cg * 16, 16)
                    fidx[staged + r, sl] = tail_v[r, sl]

        for bb in range(_NBUF):
            pltpu.async_copy(z_hbm.at[fidx.at[bb]], rows_v[bb], gsem[bb])

        def step(g, carry):
            for bb in range(_NBUF):
                t = g * _NBUF + bb

                @pl.when(t < cnt)
                def _(t=t, bb=bb):
                    pltpu.make_async_copy(z_hbm.at[fidx.at[t]],
                                          rows_v[bb], gsem[bb]).wait()

                    @pl.when(t >= _NBUF)
                    def _():
                        pltpu.make_async_copy(
                            aggr_v[bb],
                            out_hbm.at[pl.ds(0, _CHUNK_NODES)],
                            osem[bb]).wait()

                    def node_body(nl, cc, bb=bb):
                        rbase = nl * k
                        for cg in range(c // 16):
                            sl = pl.ds(cg * 16, 16)
                            acc = rows_v[bb][rbase, sl]
                            for kk in range(1, k):
                                acc = jnp.maximum(acc,
                                                  rows_v[bb][rbase + kk, sl])
                            aggr_v[bb][nl, sl] = acc
                        return cc

                    lax.fori_loop(0, _CHUNK_NODES, node_body, 0)

                    @pl.when(t + _NBUF < cnt)
                    def _():
                        pltpu.async_copy(z_hbm.at[fidx.at[t + _NBUF]],
                                         rows_v[bb], gsem[bb])

                    pltpu.async_copy(
                        aggr_v[bb],
                        out_hbm.at[pl.ds((c0 + t) * _CHUNK_NODES,
                                         _CHUNK_NODES)],
                        osem[bb])

            return carry

        lax.fori_loop(0, nouter, step, 0)

        for bb in range(_NBUF):
            pltpu.make_async_copy(aggr_v[bb],
                                  out_hbm.at[pl.ds(0, _CHUNK_NODES)],
                                  osem[bb]).wait()

    return sc_kernel(zb, e3, tail_fidx)


def _combine_kernel(aggr_ref, x0_ref, x_ref, w1_ref, w2_ref, wr_ref,
                    gamma_ref, beta_ref, bias_ref, prev_ref, out_ref):
    c = aggr_ref.shape[1]
    rows = lax.broadcasted_iota(jnp.int32, (c, c), 0)
    cols = lax.broadcasted_iota(jnp.int32, (c, c), 1)
    eye = jnp.where(rows == cols, 1.0, 0.0).astype(jnp.float32)
    w1p = w1_ref[...] * _BETA + eye * _C1
    w2p = w2_ref[...] * _BETA + eye * _C2
    cn = (((1,), (1,)), ((), ()))
    pre = lax.dot_general(aggr_ref[...], w1p, cn,
                          preferred_element_type=jnp.float32)
    pre = pre + lax.dot_general(x0_ref[...], w2p, cn,
                                preferred_element_type=jnp.float32)
    pre = pre + lax.dot_general(x_ref[...], wr_ref[...], cn,
                                preferred_element_type=jnp.float32)
    y = jnp.maximum(pre, 0.0)
    mu = jnp.mean(y, axis=1, keepdims=True)
    var = jnp.mean(y * y, axis=1, keepdims=True) - mu * mu
    yn = (y - mu) * lax.rsqrt(var + 1e-5)
    yn = yn * gamma_ref[...] + beta_ref[...]
    yb = yn + bias_ref[...]
    nrm = jnp.sqrt(jnp.sum(yb * yb, axis=1, keepdims=True))
    out_ref[...] = yb / jnp.maximum(nrm, 1e-12)


def _combine(aggr_b, x0_2d, x_nm, w1, w2, wr, gamma, beta, bias, prev,
             bw, bn, c, n):
    nblk = n // _NB
    bspec = pl.BlockSpec((_NB, c), lambda j: (j, 0))
    fspec = pl.BlockSpec((_NB, c),
                         lambda j, bw=bw, nblk=nblk: (bw * nblk + j, 0))
    w_spec = pl.BlockSpec((c, c), lambda j: (0, 0))
    v_spec = pl.BlockSpec((1, c), lambda j: (0, 0))
    pinned = pl.BlockSpec((_NB, c), lambda j: (0, 0))
    return pl.pallas_call(
        _combine_kernel,
        grid=(nblk,),
        in_specs=[bspec, fspec, fspec, w_spec, w_spec, w_spec,
                  v_spec, v_spec, v_spec, pinned],
        out_specs=fspec,
        out_shape=jax.ShapeDtypeStruct((bn, c), jnp.float32),
        input_output_aliases={9: 0},
    )(aggr_b, x0_2d, x_nm, w1, w2, wr, gamma, beta, bias, prev)


def kernel(x, x_0, edge_index, W_pre, W1, W2, Wr, ln_gamma, ln_beta, bias):
    b, c, n = x.shape[0], x.shape[1], x.shape[2]
    k = edge_index.shape[-1]
    bn = b * n
    x_nm = jnp.transpose(x[:, :, :, 0], (0, 2, 1)).reshape(bn, c)
    x0_2d = x_0.reshape(bn, c)
    e3 = jnp.transpose(edge_index, (0, 1, 3, 2)).reshape(2 * b, k, n)
    tail = edge_index[0, :, n - 16:, :]
    tail_fidx = jnp.zeros((b, 8, 128), jnp.int32).at[:, :2, :].set(
        tail.reshape(b, 2, 128))

    gamma_r = ln_gamma.reshape(1, c)
    beta_r = ln_beta.reshape(1, c)
    bias_r = bias.reshape(1, c)
    out = jnp.zeros((bn, c), jnp.float32)
    for bw in range(b):
        zb = _pre_matmul(x_nm, W_pre, bw, n, c)
        aggr_b = _sc_gather_max(zb, e3, tail_fidx, bw, c, k, n)
        out = _combine(aggr_b, x0_2d, x_nm, W1, W2, Wr,
                       gamma_r, beta_r, bias_r, out, bw, bn, c, n)
    return jnp.transpose(out.reshape(b, n, c), (0, 2, 1))[..., None]


# speedup vs baseline: 6.5821x; 1.1009x over previous
import functools
import math

import jax
import jax.numpy as jnp
from jax import lax
from jax.experimental import pallas as pl
from jax.experimental.pallas import tpu as pltpu
from jax.experimental.pallas import tpu_sc as plsc

_ALPHA = 0.1
_BETA = math.log(0.5 / 1 + 1)
_C1 = (1.0 - _ALPHA) * (1.0 - _BETA)
_C2 = _ALPHA * (1.0 - _BETA)

_CHUNK_NODES = 8
_NUM_WORKERS = 32
_NB = 2000
_NBUF = 4


def _pre_kernel(x_ref, w_ref, z_ref):
    z = lax.dot_general(x_ref[...], w_ref[...], (((1,), (1,)), ((), ())),
                        preferred_element_type=jnp.float32)
    z_ref[...] = jnp.maximum(z, 0.0)


def _pre_matmul(x_nm, w_pre, bn, c):
    return pl.pallas_call(
        _pre_kernel,
        grid=(bn // _NB,),
        in_specs=[
            pl.BlockSpec((_NB, c), lambda i: (i, 0)),
            pl.BlockSpec((c, c), lambda i: (0, 0)),
        ],
        out_specs=pl.BlockSpec((_NB, c), lambda i: (i, 0)),
        out_shape=jax.ShapeDtypeStruct((bn, c), jnp.float32),
    )(x_nm, w_pre)


def _sc_gather_max(z2d, e3, tail_fidx, bn, c, k, n):
    rows_per_chunk = _CHUNK_NODES * k
    npw = 640
    last_npw = 384
    max_cw = npw // _CHUNK_NODES
    last_cw = last_npw // _CHUNK_NODES + 2
    nouter = max_cw // _NBUF
    mesh = plsc.VectorSubcoreMesh(core_axis_name="c", subcore_axis_name="s")

    scratch = ([pltpu.VMEM((k * npw,), jnp.int32),
                pltpu.VMEM((max_cw, rows_per_chunk), jnp.int32)]
               + [pltpu.VMEM((rows_per_chunk, c), jnp.float32)] * _NBUF
               + [pltpu.VMEM((_CHUNK_NODES, c), jnp.float32)] * _NBUF
               + [pltpu.SemaphoreType.DMA] * (1 + 2 * _NBUF))

    @functools.partial(
        pl.kernel,
        mesh=mesh,
        out_type=jax.ShapeDtypeStruct((bn, c), jnp.float32),
        scratch_types=scratch,
        compiler_params=pltpu.CompilerParams(use_tc_tiling_on_sc=True,
                                             needs_layout_passes=False),
    )
    def sc_kernel(z_hbm, e_hbm, tail_hbm, out_hbm, idx_blk, fidx, *bufs):
        rows_v = bufs[:_NBUF]
        aggr_v = bufs[_NBUF:2 * _NBUF]
        isem = bufs[2 * _NBUF]
        gsem = bufs[2 * _NBUF + 1:2 * _NBUF + 1 + _NBUF]
        osem = bufs[2 * _NBUF + 1 + _NBUF:]
        wid = lax.axis_index("s") * 2 + lax.axis_index("c")
        lw = wid % 16
        bw = wid // 16
        is_last = lw == 15
        cw = jnp.where(is_last, last_cw, max_cw)
        n0 = pl.multiple_of(lw * npw, 128)
        row0 = pl.multiple_of(bw * n + lw * npw, 8)

        @pl.when(jnp.logical_not(is_last))
        def _():
            for kk in range(k):
                pltpu.async_copy(e_hbm.at[bw, kk, pl.ds(n0, npw)],
                                 idx_blk.at[pl.ds(kk * npw, npw)], isem)
            for kk in range(k):
                pltpu.make_async_copy(e_hbm.at[bw, kk, pl.ds(n0, npw)],
                                      idx_blk.at[pl.ds(kk * npw, npw)],
                                      isem).wait()

        @pl.when(is_last)
        def _():
            for kk in range(k):
                pltpu.async_copy(
                    e_hbm.at[bw, kk, pl.ds(15 * npw, last_npw)],
                    idx_blk.at[pl.ds(kk * npw, last_npw)], isem)
            for kk in range(k):
                pltpu.make_async_copy(
                    e_hbm.at[bw, kk, pl.ds(15 * npw, last_npw)],
                    idx_blk.at[pl.ds(kk * npw, last_npw)], isem).wait()

        lanes = lax.iota(jnp.int32, 16)
        jmod = jnp.where(lanes >= 8, lanes - 8, lanes)
        rowadd = jnp.where(lanes >= 8, 1, 0)
        colbase = jmod * 16
        offv = jnp.broadcast_to(bw * n, (16,)).astype(jnp.int32)

        def pair_body(p, carry):
            rows = jnp.broadcast_to(2 * p, (16,)).astype(jnp.int32) + rowadd
            for kk in range(k):
                v = idx_blk[pl.ds(kk * npw + p * 16, 16)] + offv
                plsc.store_scatter(fidx, [rows, colbase + kk], v)
            return carry

        repack_pairs = jnp.where(is_last, last_npw // 16, npw // 16)
        lax.fori_loop(0, repack_pairs, pair_body, 0)

        @pl.when(is_last)
        def _():
            pltpu.sync_copy(tail_hbm.at[bw],
                            fidx.at[pl.ds(last_npw // _CHUNK_NODES, 8)])

        for bb in range(_NBUF):
            pltpu.async_copy(z_hbm.at[fidx.at[bb]], rows_v[bb], gsem[bb])

        def step(g, carry):
            for bb in range(_NBUF):
                t = g * _NBUF + bb

                @pl.when(t < cw)
                def _(t=t, bb=bb):
                    pltpu.make_async_copy(z_hbm.at[fidx.at[t]],
                                          rows_v[bb], gsem[bb]).wait()

                    @pl.when(t >= _NBUF)
                    def _():
                        pltpu.make_async_copy(
                            aggr_v[bb],
                            out_hbm.at[pl.ds(0, _CHUNK_NODES)],
                            osem[bb]).wait()

                    def node_body(nl, cc, bb=bb):
                        rbase = nl * k
                        for cg in range(c // 16):
                            sl = pl.ds(cg * 16, 16)
                            acc = rows_v[bb][rbase, sl]
                            for kk in range(1, k):
                                acc = jnp.maximum(acc,
                                                  rows_v[bb][rbase + kk, sl])
                            aggr_v[bb][nl, sl] = acc
                        return cc

                    lax.fori_loop(0, _CHUNK_NODES, node_body, 0)

                    @pl.when(t + _NBUF < cw)
                    def _():
                        pltpu.async_copy(z_hbm.at[fidx.at[t + _NBUF]],
                                         rows_v[bb], gsem[bb])

                    pltpu.async_copy(
                        aggr_v[bb],
                        out_hbm.at[pl.ds(row0 + t * _CHUNK_NODES,
                                         _CHUNK_NODES)],
                        osem[bb])

            return carry

        lax.fori_loop(0, nouter, step, 0)

        for bb in range(_NBUF):
            pltpu.make_async_copy(aggr_v[bb],
                                  out_hbm.at[pl.ds(0, _CHUNK_NODES)],
                                  osem[bb]).wait()

    return sc_kernel(z2d, e3, tail_fidx)


def _combine_kernel(aggr_ref, x0_ref, x_ref, w1_ref, w2_ref, wr_ref,
                    gamma_ref, beta_ref, bias_ref, out_ref):
    c = aggr_ref.shape[1]
    rows = lax.broadcasted_iota(jnp.int32, (c, c), 0)
    cols = lax.broadcasted_iota(jnp.int32, (c, c), 1)
    eye = jnp.where(rows == cols, 1.0, 0.0).astype(jnp.float32)
    w1p = w1_ref[...] * _BETA + eye * _C1
    w2p = w2_ref[...] * _BETA + eye * _C2
    cn = (((1,), (1,)), ((), ()))
    pre = lax.dot_general(aggr_ref[...], w1p, cn,
                          preferred_element_type=jnp.float32)
    pre = pre + lax.dot_general(x0_ref[...], w2p, cn,
                                preferred_element_type=jnp.float32)
    pre = pre + lax.dot_general(x_ref[...], wr_ref[...], cn,
                                preferred_element_type=jnp.float32)
    y = jnp.maximum(pre, 0.0)
    mu = jnp.mean(y, axis=1, keepdims=True)
    var = jnp.mean(y * y, axis=1, keepdims=True) - mu * mu
    yn = (y - mu) * lax.rsqrt(var + 1e-5)
    yn = yn * gamma_ref[...] + beta_ref[...]
    yb = yn + bias_ref[...]
    nrm = jnp.sqrt(jnp.sum(yb * yb, axis=1, keepdims=True))
    out_ref[...] = yb / jnp.maximum(nrm, 1e-12)


def _combine(aggr, x0_2d, x_nm, w1, w2, wr, gamma, beta, bias, bn, c):
    node_spec = pl.BlockSpec((_NB, c), lambda i: (i, 0))
    w_spec = pl.BlockSpec((c, c), lambda i: (0, 0))
    v_spec = pl.BlockSpec((1, c), lambda i: (0, 0))
    return pl.pallas_call(
        _combine_kernel,
        grid=(bn // _NB,),
        in_specs=[node_spec, node_spec, node_spec, w_spec, w_spec, w_spec,
                  v_spec, v_spec, v_spec],
        out_specs=node_spec,
        out_shape=jax.ShapeDtypeStruct((bn, c), jnp.float32),
    )(aggr, x0_2d, x_nm, w1, w2, wr, gamma, beta, bias)


def kernel(x, x_0, edge_index, W_pre, W1, W2, Wr, ln_gamma, ln_beta, bias):
    b, c, n = x.shape[0], x.shape[1], x.shape[2]
    k = edge_index.shape[-1]
    bn = b * n
    x_nm = jnp.transpose(x[:, :, :, 0], (0, 2, 1)).reshape(bn, c)
    x0_2d = x_0.reshape(bn, c)
    e3 = jnp.transpose(edge_index, (0, 1, 3, 2)).reshape(2 * b, k, n)
    tail = edge_index[0, :, n - 16:, :] + (jnp.arange(b, dtype=jnp.int32)
                                           * n)[:, None, None]
    tail_fidx = jnp.zeros((b, 8, 128), jnp.int32).at[:, :2, :].set(
        tail.reshape(b, 2, 128))

    z = _pre_matmul(x_nm, W_pre, bn, c)
    aggr = _sc_gather_max(z, e3, tail_fidx, bn, c, k, n)
    out2d = _combine(aggr, x0_2d, x_nm, W1, W2, Wr,
                     ln_gamma.reshape(1, c), ln_beta.reshape(1, c),
                     bias.reshape(1, c), bn, c)
    return jnp.transpose(out2d.reshape(b, n, c), (0, 2, 1))[..., None]
